# baseline (speedup 1.0000x reference)
import jax
import jax.numpy as jnp
from jax import lax
from jax.experimental import pallas as pl
from jax.experimental.pallas import tpu as pltpu
from jax.experimental.pallas import tpu_sc as plsc

N_NODES = 10000
D = 128
N_EDGES = 320000

NC = 2
NS = 16
NW = NC * NS

CH = 128
NITER = 80
NCH_TOT = NW * NITER
E_PAD = NCH_TOT * CH
NB = 2
NPH = 2
PH = NITER // NPH

RPT = 632
NPAD = NS * RPT
DEG_E = PH * CH
DSTAGE = 1280

_f32 = jnp.float32
_mesh = plsc.VectorSubcoreMesh(core_axis_name="c", subcore_axis_name="s")


def _make_sc_agg(compute_deg: bool):
    out_type = [jax.ShapeDtypeStruct((NC, NPAD, D), _f32)]
    if compute_deg:
        out_type.append(jax.ShapeDtypeStruct((NC * NPAD,), _f32))

    scratch = [
        pltpu.VMEM((PH, CH), jnp.int32),
        pltpu.VMEM((PH, CH), jnp.int32),
        pltpu.VMEM((NB, CH, D), _f32),
        pltpu.VMEM_SHARED((NPAD, D), _f32),
    ] + [pltpu.SemaphoreType.DMA] * (2 * NB)
    if compute_deg:
        scratch += [
            pltpu.VMEM((PH, CH), _f32),
            pltpu.VMEM((DSTAGE,), _f32),
            pltpu.VMEM_SHARED((NPAD,), _f32),
            pltpu.SemaphoreType.DMA,
        ]

    def body(x_hbm, src_hbm, dst_hbm, z2d_hbm, z1d_hbm, *rest):
        if compute_deg:
            o2d_hbm = rest[0]
            (agg_out, deg_out, src_v, dst_v, rows_v, agg_sh,
             *sems) = rest[1:7 + 2 * NB]
            ones_v, deg_v, deg_sh, dsem = rest[7 + 2 * NB:]
        else:
            (agg_out, src_v, dst_v, rows_v, agg_sh, *sems) = rest
        gsem = sems[:NB]
        ssem = sems[NB:2 * NB]

        cid = lax.axis_index("c")
        sid = lax.axis_index("s")
        wid = cid * NS + sid

        pltpu.sync_copy(z2d_hbm.at[pl.ds(sid * RPT, RPT)],
                        agg_sh.at[pl.ds(sid * RPT, RPT)])

        if compute_deg:
            pltpu.sync_copy(o2d_hbm, ones_v)

            @pl.when(sid == 0)
            def _():
                pltpu.sync_copy(z1d_hbm, deg_v)
                for t in range(NPAD // DSTAGE):
                    pltpu.sync_copy(deg_v,
                                    deg_sh.at[pl.ds(t * DSTAGE, DSTAGE)])
                rem = NPAD % DSTAGE
                pltpu.sync_copy(
                    deg_v.at[pl.ds(0, rem)],
                    deg_sh.at[pl.ds(NPAD - rem, rem)])

        plsc.subcore_barrier()

        def wait_bytes(sem, like):
            pltpu.make_async_copy(x_hbm.at[pl.ds(0, CH)], like, sem).wait()

        def wait_deg(sem):
            for _ in range(DEG_E // DSTAGE):
                pltpu.make_async_copy(z1d_hbm, deg_v, sem).wait()

        def step(i, b, refill):
            wait_bytes(gsem[b], rows_v.at[b])
            pltpu.async_copy(rows_v.at[b], agg_sh.at[dst_v.at[i]],
                             ssem[b], add=True)
            if compute_deg:
                pltpu.async_copy(ones_v.at[0], deg_sh.at[dst_v.at[i]],
                                 dsem, add=True)
            if refill:
                wait_bytes(ssem[b], rows_v.at[b])
                pltpu.async_copy(x_hbm.at[src_v.at[i + NB]],
                                 rows_v.at[b], gsem[b])

        for ph in range(NPH):
            if ph > 0:
                for b in range(NB):
                    wait_bytes(ssem[b], rows_v.at[b])
                if compute_deg:
                    wait_deg(dsem)

            base = wid * NITER + ph * PH
            pltpu.sync_copy(src_hbm.at[pl.ds(base, PH)], src_v)
            pltpu.sync_copy(dst_hbm.at[pl.ds(base, PH)], dst_v)

            for b in range(NB):
                pltpu.async_copy(x_hbm.at[src_v.at[b]], rows_v.at[b],
                                 gsem[b])

            def group_body(p, carry):
                i0 = p * NB
                for b in range(NB):
                    step(i0 + b, b, refill=True)
                return carry

            lax.fori_loop(0, PH // NB - 1, group_body, 0)
            for b in range(NB):
                step(PH - NB + b, b, refill=False)

        for b in range(NB):
            wait_bytes(ssem[b], rows_v.at[b])
        if compute_deg:
            wait_deg(dsem)

        plsc.subcore_barrier()

        pltpu.sync_copy(agg_sh.at[pl.ds(sid * RPT, RPT)],
                        agg_out.at[cid, pl.ds(sid * RPT, RPT)])
        if compute_deg:
            @pl.when(sid == 0)
            def _():
                for t in range(NPAD // DSTAGE):
                    pltpu.sync_copy(deg_sh.at[pl.ds(t * DSTAGE, DSTAGE)],
                                    deg_v)
                    pltpu.sync_copy(
                        deg_v,
                        deg_out.at[pl.ds(cid * NPAD + t * DSTAGE, DSTAGE)])
                rem = NPAD % DSTAGE
                pltpu.sync_copy(deg_sh.at[pl.ds(NPAD - rem, rem)],
                                deg_v.at[pl.ds(0, rem)])
                pltpu.sync_copy(
                    deg_v.at[pl.ds(0, rem)],
                    deg_out.at[pl.ds(cid * NPAD + NPAD - rem, rem)])

    return pl.kernel(
        body, mesh=_mesh, out_type=tuple(out_type), scratch_types=scratch)


_sc_agg_deg = _make_sc_agg(True)
_sc_agg = _make_sc_agg(False)


BLK = 1000


def _dot_t(a, w):
    return lax.dot_general(a, w, (((1,), (1,)), ((), ())),
                           preferred_element_type=_f32)


def _tc_self(x_ref, w_ref, b_ref, out_ref):
    out_ref[...] = _dot_t(x_ref[...], w_ref[...]) + b_ref[...]


def _tc_agg1(aggp_ref, d0_ref, d1_ref, wl_ref, self_ref, h_ref, inv_ref):
    inv = 1.0 / jnp.maximum(d0_ref[...] + d1_ref[...], 1.0)
    agg = (aggp_ref[0] + aggp_ref[1]) * inv
    h_ref[...] = jnp.maximum(_dot_t(agg, wl_ref[...]) + self_ref[...], 0.0)
    inv_ref[...] = inv


def _tc_agg2(aggp_ref, inv_ref, wl_ref, self_ref, out_ref):
    agg = (aggp_ref[0] + aggp_ref[1]) * inv_ref[...]
    out_ref[...] = _dot_t(agg, wl_ref[...]) + self_ref[...]


_w_spec = pl.BlockSpec((D, D), lambda i: (0, 0))
_b_spec = pl.BlockSpec((1, D), lambda i: (0, 0))
_aggp_spec = pl.BlockSpec((NC, BLK, D), lambda i: (0, i, 0))
_row_spec = pl.BlockSpec((BLK, D), lambda i: (i, 0))
_col_spec = pl.BlockSpec((BLK, 1), lambda i: (i, 0))
_grid = (N_NODES // BLK,)
_row_shape = jax.ShapeDtypeStruct((N_NODES, D), _f32)
_col_shape = jax.ShapeDtypeStruct((N_NODES, 1), _f32)


def _tc_self_call(x, w, b):
    return pl.pallas_call(
        _tc_self, grid=_grid,
        in_specs=[_row_spec, _w_spec, _b_spec],
        out_specs=_row_spec, out_shape=_row_shape,
    )(x, w, b)


def _tc_agg1_call(aggp, d0, d1, wl, selfterm):
    return pl.pallas_call(
        _tc_agg1, grid=_grid,
        in_specs=[_aggp_spec, _col_spec, _col_spec, _w_spec, _row_spec],
        out_specs=[_row_spec, _col_spec],
        out_shape=[_row_shape, _col_shape],
    )(aggp, d0, d1, wl, selfterm)


def _tc_agg2_call(aggp, inv, wl, selfterm):
    return pl.pallas_call(
        _tc_agg2, grid=_grid,
        in_specs=[_aggp_spec, _col_spec, _w_spec, _row_spec],
        out_specs=_row_spec, out_shape=_row_shape,
    )(aggp, inv, wl, selfterm)


@jax.jit
def kernel(x, edge_index, W1_l, b1_l, W1_r, W2_l, b2_l, W2_r):
    x = x.astype(_f32)
    src = edge_index[0].astype(jnp.int32)
    dst = edge_index[1].astype(jnp.int32)

    n_pad = E_PAD - N_EDGES
    pad_ar = jnp.arange(n_pad, dtype=jnp.int32)
    src_pad = (pad_ar * 131) % N_NODES
    dst_pad = N_NODES + pad_ar % (NPAD - N_NODES)
    src2d = jnp.concatenate([src, src_pad]).reshape(NCH_TOT, CH)
    dst2d = jnp.concatenate([dst, dst_pad]).reshape(NCH_TOT, CH)

    z2d = jnp.zeros((NPAD, D), _f32)
    z1d = jnp.zeros((DSTAGE,), _f32)
    o2d = jnp.ones((PH, CH), _f32)

    aggp1, deg_flat = _sc_agg_deg(x, src2d, dst2d, z2d, z1d, o2d)
    d0 = deg_flat[:N_NODES].reshape(N_NODES, 1)
    d1 = deg_flat[NPAD:NPAD + N_NODES].reshape(N_NODES, 1)

    self1 = _tc_self_call(x, W1_r, b1_l.reshape(1, D))
    h, inv = _tc_agg1_call(aggp1, d0, d1, W1_l, self1)

    (aggp2,) = _sc_agg(h, src2d, dst2d, z2d, z1d)
    self2 = _tc_self_call(h, W2_r, b2_l.reshape(1, D))
    out = _tc_agg2_call(aggp2, inv, W2_l, self2)
    return out
